# Initial kernel scaffold; baseline (speedup 1.0000x reference)
import jax
import jax.numpy as jnp
from jax.experimental import pallas as pl


def kernel(seq_output, hidden, transformer_output, index):
    raise NotImplementedError("write your pallas kernel here")


# speedup vs baseline: 2.3657x; 2.3657x over previous
import jax
import jax.numpy as jnp
from jax import lax
from jax.experimental import pallas as pl
from jax.experimental.pallas import tpu as pltpu
from jax.experimental.pallas import tpu_sc as plsc

B, S, DS = 8, 2048, 128
N_NODES, IDX_NUM, DG = 512, 4, 128
NC, NS, L = 2, 16, 16
SAMPLES_PER_SC = B // NC
CHUNKS = NS // SAMPLES_PER_SC
NODES_PER_CHUNK = N_NODES // CHUNKS
ROWS_PER_TILE = S // CHUNKS
BLK = 128


def _sc_body(t_hbm, idx_hbm, seq_hbm, out_hbm,
             acc_s, cnt_s,
             g_l, idx_l, ones_l, out_l):
    c = lax.axis_index("c")
    s = lax.axis_index("s")
    b = c * SAMPLES_PER_SC + s // CHUNKS
    b_loc = s // CHUNKS
    chunk = s % CHUNKS
    dest_base = b_loc * S + chunk * ROWS_PER_TILE

    zero16 = jnp.zeros((L,), jnp.float32)
    one16 = jnp.ones((L,), jnp.float32)

    @pl.loop(0, BLK)
    def _(r):
        ones_l[r, :] = one16
        for kk in range(DG // L):
            out_l[r, pl.ds(kk * L, L)] = zero16

    for k in range(ROWS_PER_TILE // BLK):
        pltpu.sync_copy(out_l.at[:, pl.ds(0, DG)],
                        acc_s.at[pl.ds(dest_base + k * BLK, BLK)])
        pltpu.sync_copy(out_l.at[:, pl.ds(0, L)],
                        cnt_s.at[pl.ds(dest_base + k * BLK, BLK)])

    pltpu.sync_copy(t_hbm.at[b, pl.ds(chunk * NODES_PER_CHUNK, NODES_PER_CHUNK)], g_l)
    pltpu.sync_copy(idx_hbm.at[b, chunk], idx_l)

    plsc.subcore_barrier()

    for i in range(IDX_NUM):
        pltpu.sync_copy(g_l, acc_s.at[idx_l.at[i]], add=True)
        pltpu.sync_copy(ones_l, cnt_s.at[idx_l.at[i]], add=True)

    plsc.subcore_barrier()

    for k in range(ROWS_PER_TILE // BLK):
        rows = dest_base + k * BLK
        s0 = chunk * ROWS_PER_TILE + k * BLK
        pltpu.sync_copy(acc_s.at[pl.ds(rows, BLK)], g_l)
        pltpu.sync_copy(cnt_s.at[pl.ds(rows, BLK)], ones_l)
        pltpu.sync_copy(seq_hbm.at[b, pl.ds(s0, BLK)],
                        out_l.at[:, pl.ds(DG, DS)])

        @pl.loop(0, BLK)
        def _(r):
            inv = 1.0 / jnp.maximum(ones_l[r, :], 1.0)
            for kk in range(DG // L):
                out_l[r, pl.ds(kk * L, L)] = (
                    g_l[r, pl.ds(kk * L, L)] + 1e-8) * inv

        pltpu.sync_copy(out_l, out_hbm.at[b, pl.ds(s0, BLK)])


@jax.jit
def _sc_call(t, idx_p, seq):
    mesh = plsc.VectorSubcoreMesh(core_axis_name="c", subcore_axis_name="s",
                                  num_cores=NC, num_subcores=NS)
    return pl.kernel(
        _sc_body,
        out_type=jax.ShapeDtypeStruct((B, S, DG + DS), jnp.float32),
        mesh=mesh,
        compiler_params=pltpu.CompilerParams(use_tc_tiling_on_sc=False),
        scratch_types=[
            pltpu.VMEM_SHARED((SAMPLES_PER_SC * S, DG), jnp.float32),
            pltpu.VMEM_SHARED((SAMPLES_PER_SC * S, L), jnp.float32),
            pltpu.VMEM((NODES_PER_CHUNK, DG), jnp.float32),
            pltpu.VMEM((IDX_NUM, NODES_PER_CHUNK), jnp.int32),
            pltpu.VMEM((NODES_PER_CHUNK, L), jnp.float32),
            pltpu.VMEM((BLK, DG + DS), jnp.float32),
        ],
    )(t, idx_p, seq)


def kernel(seq_output, hidden, transformer_output, index):
    idx_p = index.astype(jnp.int32).reshape(B, CHUNKS, NODES_PER_CHUNK, IDX_NUM)
    idx_p = idx_p.transpose(0, 1, 3, 2)
    offs = (jnp.arange(B, dtype=jnp.int32) % SAMPLES_PER_SC) * S
    idx_p = idx_p + offs[:, None, None, None]

    enc_output = _sc_call(transformer_output, idx_p, seq_output)
    hidden_flat = hidden.reshape(hidden.shape[0], -1)
    return (enc_output, hidden_flat)


# speedup vs baseline: 3.2209x; 1.3615x over previous
import jax
import jax.numpy as jnp
from jax import lax
from jax.experimental import pallas as pl
from jax.experimental.pallas import tpu as pltpu
from jax.experimental.pallas import tpu_sc as plsc

B, S, DS = 8, 2048, 128
N_NODES, IDX_NUM, DG = 512, 4, 128
NC, NS, L = 2, 16, 16
SAMPLES_PER_SC = B // NC
CHUNKS = NS // SAMPLES_PER_SC
NODES_PER_CHUNK = N_NODES // CHUNKS
ROWS_PER_TILE = S // CHUNKS
BLK = 128
RT = BLK // 8


def _sc_body(t_hbm, idx_hbm, seq_hbm, out_hbm,
             acc_s, cnt_s,
             g_l, idx_l, ones_l, out_l, sem_in, sem_sc, sem_ld, sem_st):
    c = lax.axis_index("c")
    s = lax.axis_index("s")
    b = c * SAMPLES_PER_SC + s // CHUNKS
    b_loc = s // CHUNKS
    chunk = s % CHUNKS
    dest_base = b_loc * S + chunk * ROWS_PER_TILE

    zero16 = jnp.zeros((L,), jnp.float32)
    one16 = jnp.ones((L,), jnp.float32)

    @pl.loop(0, BLK)
    def _(r):
        ones_l[r, :] = zero16
        for kk in range(DG // L):
            g_l[r, pl.ds(kk * L, L)] = zero16

    in2 = pltpu.async_copy(idx_hbm.at[b, chunk], idx_l, sem_in)

    inits = []
    for k in range(ROWS_PER_TILE // BLK):
        inits.append(pltpu.async_copy(
            g_l, acc_s.at[pl.ds(dest_base + k * BLK, BLK)], sem_ld))
        inits.append(pltpu.async_copy(
            ones_l, cnt_s.at[pl.ds(dest_base + k * BLK, BLK)], sem_ld))
    for d in inits:
        d.wait()

    in1 = pltpu.async_copy(
        t_hbm.at[b, pl.ds(chunk * NODES_PER_CHUNK, NODES_PER_CHUNK)], g_l,
        sem_in)

    @pl.loop(0, BLK)
    def _(r):
        ones_l[r, :] = one16

    in1.wait()
    in2.wait()

    plsc.subcore_barrier()

    scats = []
    for i in range(IDX_NUM):
        scats.append(pltpu.async_copy(
            g_l, acc_s.at[idx_l.at[i]], sem_sc, add=True))
        scats.append(pltpu.async_copy(
            ones_l, cnt_s.at[idx_l.at[i]], sem_sc, add=True))
    for d in scats:
        d.wait()

    plsc.subcore_barrier()

    nblk = ROWS_PER_TILE // BLK

    def start_acc_loads(k):
        rows = dest_base + k * BLK
        return (
            pltpu.async_copy(acc_s.at[pl.ds(rows, BLK)], g_l, sem_ld),
            pltpu.async_copy(cnt_s.at[pl.ds(rows, BLK)], ones_l, sem_ld),
        )

    def start_seq_load(k):
        s0 = chunk * ROWS_PER_TILE + k * BLK
        return pltpu.async_copy(seq_hbm.at[b, pl.ds(s0 // 8, RT)],
                                out_l.at[:, 1], sem_in)

    loads = start_acc_loads(0)
    seq_ld = start_seq_load(0)
    st = None
    for k in range(nblk):
        for d in loads:
            d.wait()
        seq_ld.wait()

        @pl.loop(0, BLK)
        def _(r):
            inv = 1.0 / jnp.maximum(ones_l[r, :], 1.0)
            for kk in range(DG // L):
                out_l[r // 8, 0, r % 8, pl.ds(kk * L, L)] = (
                    g_l[r, pl.ds(kk * L, L)] + 1e-8) * inv

        s0 = chunk * ROWS_PER_TILE + k * BLK
        st = pltpu.async_copy(out_l, out_hbm.at[b, pl.ds(s0 // 8, RT)], sem_st)
        if k + 1 < nblk:
            loads = start_acc_loads(k + 1)
            st.wait()
            seq_ld = start_seq_load(k + 1)
        else:
            st.wait()


@jax.jit
def _sc_call(t, idx_p, seq5):
    mesh = plsc.VectorSubcoreMesh(core_axis_name="c", subcore_axis_name="s",
                                  num_cores=NC, num_subcores=NS)
    return pl.kernel(
        _sc_body,
        out_type=jax.ShapeDtypeStruct((B, S // 8, 2, 8, DG), jnp.float32),
        mesh=mesh,
        compiler_params=pltpu.CompilerParams(use_tc_tiling_on_sc=False),
        scratch_types=[
            pltpu.VMEM_SHARED((SAMPLES_PER_SC * S, DG), jnp.float32),
            pltpu.VMEM_SHARED((SAMPLES_PER_SC * S, L), jnp.float32),
            pltpu.VMEM((BLK, DG), jnp.float32),
            pltpu.VMEM((IDX_NUM, NODES_PER_CHUNK), jnp.int32),
            pltpu.VMEM((BLK, L), jnp.float32),
            pltpu.VMEM((RT, 2, 8, DG), jnp.float32),
            pltpu.SemaphoreType.DMA,
            pltpu.SemaphoreType.DMA,
            pltpu.SemaphoreType.DMA,
            pltpu.SemaphoreType.DMA,
        ],
    )(t, idx_p, seq5)


def kernel(seq_output, hidden, transformer_output, index):
    idx_p = index.astype(jnp.int32).reshape(B, CHUNKS, NODES_PER_CHUNK, IDX_NUM)
    idx_p = idx_p.transpose(0, 1, 3, 2)
    offs = (jnp.arange(B, dtype=jnp.int32) % SAMPLES_PER_SC) * S
    idx_p = idx_p + offs[:, None, None, None]

    seq5 = seq_output.reshape(B, S // 8, 8, DS)
    out5 = _sc_call(transformer_output, idx_p, seq5)
    enc_output = out5.transpose(0, 1, 3, 2, 4).reshape(B, S, DG + DS)
    hidden_flat = hidden.reshape(hidden.shape[0], -1)
    return (enc_output, hidden_flat)


# speedup vs baseline: 4.3892x; 1.3627x over previous
import jax
import jax.numpy as jnp
from jax import lax
from jax.experimental import pallas as pl
from jax.experimental.pallas import tpu as pltpu
from jax.experimental.pallas import tpu_sc as plsc

B, S, DS = 8, 2048, 128
N_NODES, IDX_NUM, DG = 512, 4, 128
NC, NS, L = 2, 16, 16
SAMPLES_PER_SC = B // NC
CHUNKS = NS // SAMPLES_PER_SC
NODES_PER_CHUNK = N_NODES // CHUNKS
ROWS_PER_TILE = S // CHUNKS
HBLK = 64
HRT = HBLK // 8
NBLK = ROWS_PER_TILE // HBLK


def _sc_body(t_hbm, idx_hbm, seq_hbm, out_hbm,
             acc_s, cnt_s,
             g_l, idx_l, ones_l, gout_l, seq_l,
             sem_in, sem_sc, sem_ld0, sem_ld1, sem_sq0, sem_sq1,
             sem_gs0, sem_gs1, sem_ss0, sem_ss1):
    c = lax.axis_index("c")
    s = lax.axis_index("s")
    b = c * SAMPLES_PER_SC + s // CHUNKS
    b_loc = s // CHUNKS
    chunk = s % CHUNKS
    dest_base = b_loc * S + chunk * ROWS_PER_TILE

    sem_ld = (sem_ld0, sem_ld1)
    sem_sq = (sem_sq0, sem_sq1)
    sem_gs = (sem_gs0, sem_gs1)
    sem_ss = (sem_ss0, sem_ss1)

    zero16 = jnp.zeros((L,), jnp.float32)
    one16 = jnp.ones((L,), jnp.float32)
    eps16 = jnp.full((L,), 1e-8, jnp.float32)

    in2 = pltpu.async_copy(idx_hbm.at[b, chunk], idx_l, sem_in)

    def start_seq_load(k):
        s0 = chunk * ROWS_PER_TILE + k * HBLK
        j = k % 2
        return pltpu.async_copy(seq_hbm.at[b, pl.ds(s0 // 8, HRT)],
                                seq_l.at[pl.ds(j * HRT, HRT)], sem_sq[j])

    seq_d = [start_seq_load(0), start_seq_load(1)]

    @plsc.parallel_loop(0, NODES_PER_CHUNK)
    def _(r):
        ones_l[r, :] = zero16
        for kk in range(DG // L):
            g_l[r, pl.ds(kk * L, L)] = eps16

    inits = []
    for k in range(ROWS_PER_TILE // NODES_PER_CHUNK):
        inits.append(pltpu.async_copy(
            g_l, acc_s.at[pl.ds(dest_base + k * NODES_PER_CHUNK,
                                NODES_PER_CHUNK)], sem_ld0))
        inits.append(pltpu.async_copy(
            ones_l, cnt_s.at[pl.ds(dest_base + k * NODES_PER_CHUNK,
                                   NODES_PER_CHUNK)], sem_ld1))
    for d in inits:
        d.wait()

    in1 = pltpu.async_copy(
        t_hbm.at[b, pl.ds(chunk * NODES_PER_CHUNK, NODES_PER_CHUNK)], g_l,
        sem_in)

    @plsc.parallel_loop(0, NODES_PER_CHUNK)
    def _(r):
        ones_l[r, :] = one16

    in1.wait()
    in2.wait()

    plsc.subcore_barrier()

    scats = []
    for i in range(IDX_NUM):
        scats.append(pltpu.async_copy(
            g_l, acc_s.at[idx_l.at[i]], sem_sc, add=True))
        scats.append(pltpu.async_copy(
            ones_l, cnt_s.at[idx_l.at[i]], sem_sc, add=True))
    for d in scats:
        d.wait()

    plsc.subcore_barrier()

    def start_acc_loads(k):
        rows = dest_base + k * HBLK
        j = k % 2
        return (
            pltpu.async_copy(acc_s.at[pl.ds(rows, HBLK)],
                             g_l.at[pl.ds(j * HBLK, HBLK)], sem_ld[j]),
            pltpu.async_copy(cnt_s.at[pl.ds(rows, HBLK)],
                             ones_l.at[pl.ds(j * HBLK, HBLK)], sem_ld[j]),
        )

    ld_d = [start_acc_loads(0), start_acc_loads(1)]
    st_g = [None, None]
    st_s = [None, None]
    for k in range(NBLK):
        j = k % 2
        for d in ld_d[j]:
            d.wait()
        seq_d[j].wait()
        s0 = chunk * ROWS_PER_TILE + k * HBLK
        st_s[j] = pltpu.async_copy(seq_l.at[pl.ds(j * HRT, HRT)],
                                   out_hbm.at[b, pl.ds(s0 // 8, HRT), 1],
                                   sem_ss[j])
        if st_g[j] is not None:
            st_g[j].wait()

        @plsc.parallel_loop(0, HRT)
        def _(rt):
            for r8 in range(8):
                r = j * HBLK + rt * 8 + r8
                inv = 1.0 / jnp.maximum(ones_l[r, :], 1.0)
                for kk in range(DG // L):
                    gout_l[j * HRT + rt, r8, pl.ds(kk * L, L)] = (
                        g_l[r, pl.ds(kk * L, L)] * inv)

        st_g[j] = pltpu.async_copy(gout_l.at[pl.ds(j * HRT, HRT)],
                                   out_hbm.at[b, pl.ds(s0 // 8, HRT), 0],
                                   sem_gs[j])
        if k + 2 < NBLK:
            ld_d[j] = start_acc_loads(k + 2)
        if 1 <= k < NBLK - 1:
            st_s[1 - j].wait()
            seq_d[1 - j] = start_seq_load(k + 1)

    st_g[0].wait()
    st_g[1].wait()
    st_s[0].wait()
    st_s[1].wait()


@jax.jit
def _sc_call(t, idx_p, seq5):
    mesh = plsc.VectorSubcoreMesh(core_axis_name="c", subcore_axis_name="s",
                                  num_cores=NC, num_subcores=NS)
    return pl.kernel(
        _sc_body,
        out_type=jax.ShapeDtypeStruct((B, S // 8, 2, 8, DG), jnp.float32),
        mesh=mesh,
        compiler_params=pltpu.CompilerParams(use_tc_tiling_on_sc=False),
        scratch_types=[
            pltpu.VMEM_SHARED((SAMPLES_PER_SC * S, DG), jnp.float32),
            pltpu.VMEM_SHARED((SAMPLES_PER_SC * S, L), jnp.float32),
            pltpu.VMEM((NODES_PER_CHUNK, DG), jnp.float32),
            pltpu.VMEM((IDX_NUM, NODES_PER_CHUNK), jnp.int32),
            pltpu.VMEM((NODES_PER_CHUNK, L), jnp.float32),
            pltpu.VMEM((2 * HRT, 8, DG), jnp.float32),
            pltpu.VMEM((2 * HRT, 8, DS), jnp.float32),
        ] + [pltpu.SemaphoreType.DMA] * 10,
    )(t, idx_p, seq5)


def kernel(seq_output, hidden, transformer_output, index):
    idx_p = index.astype(jnp.int32).reshape(B, CHUNKS, NODES_PER_CHUNK, IDX_NUM)
    idx_p = idx_p.transpose(0, 1, 3, 2)
    offs = (jnp.arange(B, dtype=jnp.int32) % SAMPLES_PER_SC) * S
    idx_p = idx_p + offs[:, None, None, None]

    seq5 = seq_output.reshape(B, S // 8, 8, DS)
    out5 = _sc_call(transformer_output, idx_p, seq5)
    enc_output = out5.transpose(0, 1, 3, 2, 4).reshape(B, S, DG + DS)
    hidden_flat = hidden.reshape(hidden.shape[0], -1)
    return (enc_output, hidden_flat)
